# baseline (speedup 1.0000x reference)
import jax
import jax.numpy as jnp
from jax import lax
from jax.experimental import pallas as pl
from jax.experimental.pallas import tpu as pltpu
from jax.experimental.pallas import tpu_sc as plsc

BATCH = 16384
DIM = 64
NUM_CORES = 2
NUM_SUBCORES = 16
NUM_WORKERS = NUM_CORES * NUM_SUBCORES
B_PER_W = BATCH // NUM_WORKERS
IDX_CHUNK = 128
N_CHUNKS = B_PER_W // IDX_CHUNK
LANES = 16
PACKED = 2 * LANES
D_CHUNKS = DIM // PACKED
ROW_GROUPS = B_PER_W // LANES


def _bpr_body(user_idx_hbm, item_idx_hbm, user_table_hbm, item_table_hbm,
              out_hbm, idx_u, idx_i, u_rows, i_rows, out_v, sem):
    wid = lax.axis_index("s") * NUM_CORES + lax.axis_index("c")
    base = wid * B_PER_W

    for j in range(N_CHUNKS):
        off = base + j * IDX_CHUNK
        pltpu.sync_copy(user_idx_hbm.at[pl.ds(off, IDX_CHUNK)], idx_u.at[j])
        pltpu.sync_copy(item_idx_hbm.at[pl.ds(off, IDX_CHUNK)], idx_i.at[j])

    copies = []
    for j in range(N_CHUNKS):
        dst = u_rows.at[pl.ds(j * IDX_CHUNK, IDX_CHUNK)]
        copies.append(pltpu.async_copy(user_table_hbm.at[idx_u.at[j]], dst, sem))
        dst = i_rows.at[pl.ds(j * IDX_CHUNK, IDX_CHUNK)]
        copies.append(pltpu.async_copy(item_table_hbm.at[idx_i.at[j]], dst, sem))
    for c in copies:
        c.wait()

    lane = lax.broadcasted_iota(jnp.int32, (LANES,), 0)
    perms = [lane ^ sh for sh in (8, 4, 2, 1)]

    def dot_chunk(u_ref, i_ref, r, d):
        up = u_ref[r, pl.ds(d * PACKED, PACKED)]
        ip = i_ref[r, pl.ds(d * PACKED, PACKED)]
        ua, ub = plsc.unpack(up, format=plsc.PackFormat.INTERLEAVED)
        ia, ib = plsc.unpack(ip, format=plsc.PackFormat.INTERLEAVED)
        return ua * ia + ub * ib

    def group(g, carry):
        acc = jnp.zeros((LANES,), jnp.float32)
        for j in range(LANES):
            r = g * LANES + j
            s = dot_chunk(u_rows, i_rows, r, 0)
            for d in range(1, D_CHUNKS):
                s = s + dot_chunk(u_rows, i_rows, r, d)
            for q in perms:
                s = s + s.at[q].get(mode="promise_in_bounds")
            acc = jnp.where(lane == j, s, acc)
        out_v[pl.ds(g * LANES, LANES)] = acc
        return carry

    lax.fori_loop(0, ROW_GROUPS, group, 0)

    pltpu.sync_copy(out_v, out_hbm.at[pl.ds(base, B_PER_W)])


@jax.jit
def kernel(user_idx, item_idx, user_table, item_table):
    ut_bf = user_table.astype(jnp.bfloat16)
    it_bf = item_table.astype(jnp.bfloat16)
    mesh = plsc.VectorSubcoreMesh(core_axis_name="c", subcore_axis_name="s",
                                  num_cores=NUM_CORES, num_subcores=NUM_SUBCORES)
    run = pl.kernel(
        _bpr_body,
        out_type=jax.ShapeDtypeStruct((BATCH,), jnp.float32),
        mesh=mesh,
        compiler_params=pltpu.CompilerParams(use_tc_tiling_on_sc=False,
                                             needs_layout_passes=False),
        scratch_types=[
            pltpu.VMEM((N_CHUNKS, IDX_CHUNK), jnp.int32),
            pltpu.VMEM((N_CHUNKS, IDX_CHUNK), jnp.int32),
            pltpu.VMEM((B_PER_W, DIM), jnp.bfloat16),
            pltpu.VMEM((B_PER_W, DIM), jnp.bfloat16),
            pltpu.VMEM((B_PER_W,), jnp.float32),
            pltpu.SemaphoreType.DMA,
        ],
    )
    return run(user_idx, item_idx, ut_bf, it_bf)


# speedup vs baseline: 1.3118x; 1.3118x over previous
import jax
import jax.numpy as jnp
from jax import lax
from jax.experimental import pallas as pl
from jax.experimental.pallas import tpu as pltpu
from jax.experimental.pallas import tpu_sc as plsc

BATCH = 16384
DIM = 64
NUM_CORES = 2
NUM_SUBCORES = 16
NUM_WORKERS = NUM_CORES * NUM_SUBCORES
B_PER_W = BATCH // NUM_WORKERS
IDX_CHUNK = 128
N_CHUNKS = B_PER_W // IDX_CHUNK
LANES = 16
D_CHUNKS = DIM // LANES
ROW_GROUPS = B_PER_W // LANES

_MESH = plsc.VectorSubcoreMesh(core_axis_name="c", subcore_axis_name="s",
                               num_cores=NUM_CORES, num_subcores=NUM_SUBCORES)
_PARAMS = pltpu.CompilerParams(use_tc_tiling_on_sc=False,
                               needs_layout_passes=False)


def _gather_body(idx_hbm, table_hbm, out_hbm, idx_v, rows_v, sem):
    wid = lax.axis_index("s") * NUM_CORES + lax.axis_index("c")
    base = wid * B_PER_W
    for j in range(N_CHUNKS):
        pltpu.sync_copy(idx_hbm.at[pl.ds(base + j * IDX_CHUNK, IDX_CHUNK)],
                        idx_v.at[j])
    copies = []
    for j in range(N_CHUNKS):
        dst = rows_v.at[pl.ds(j * IDX_CHUNK, IDX_CHUNK)]
        copies.append(pltpu.async_copy(table_hbm.at[idx_v.at[j]], dst, sem))
    for c in copies:
        c.wait()
    pltpu.sync_copy(rows_v, out_hbm.at[pl.ds(base, B_PER_W)])


def _gather_dot_body(idx_hbm, table_hbm, urows_hbm, out_hbm,
                     idx_v, i_rows, u_rows, out_v, sem):
    wid = lax.axis_index("s") * NUM_CORES + lax.axis_index("c")
    base = wid * B_PER_W
    for j in range(N_CHUNKS):
        pltpu.sync_copy(idx_hbm.at[pl.ds(base + j * IDX_CHUNK, IDX_CHUNK)],
                        idx_v.at[j])
    copies = [pltpu.async_copy(urows_hbm.at[pl.ds(base, B_PER_W)], u_rows, sem)]
    for j in range(N_CHUNKS):
        dst = i_rows.at[pl.ds(j * IDX_CHUNK, IDX_CHUNK)]
        copies.append(pltpu.async_copy(table_hbm.at[idx_v.at[j]], dst, sem))
    for c in copies:
        c.wait()

    lane = lax.broadcasted_iota(jnp.int32, (LANES,), 0)
    perms = [lane ^ sh for sh in (8, 4, 2, 1)]

    def group(g, carry):
        acc = jnp.zeros((LANES,), jnp.float32)
        for j in range(LANES):
            r = g * LANES + j
            s = u_rows[r, pl.ds(0, LANES)] * i_rows[r, pl.ds(0, LANES)]
            for d in range(1, D_CHUNKS):
                s = s + (u_rows[r, pl.ds(d * LANES, LANES)]
                         * i_rows[r, pl.ds(d * LANES, LANES)])
            for q in perms:
                s = s + s.at[q].get(mode="promise_in_bounds")
            acc = jnp.where(lane == j, s, acc)
        out_v[pl.ds(g * LANES, LANES)] = acc
        return carry

    lax.fori_loop(0, ROW_GROUPS, group, 0)
    pltpu.sync_copy(out_v, out_hbm.at[pl.ds(base, B_PER_W)])


@jax.jit
def kernel(user_idx, item_idx, user_table, item_table):
    gather_u = pl.kernel(
        _gather_body,
        out_type=jax.ShapeDtypeStruct((BATCH, DIM), jnp.float32),
        mesh=_MESH,
        compiler_params=_PARAMS,
        scratch_types=[
            pltpu.VMEM((N_CHUNKS, IDX_CHUNK), jnp.int32),
            pltpu.VMEM((B_PER_W, DIM), jnp.float32),
            pltpu.SemaphoreType.DMA,
        ],
    )
    u_rows = gather_u(user_idx, user_table)

    gather_dot = pl.kernel(
        _gather_dot_body,
        out_type=jax.ShapeDtypeStruct((BATCH,), jnp.float32),
        mesh=_MESH,
        compiler_params=_PARAMS,
        scratch_types=[
            pltpu.VMEM((N_CHUNKS, IDX_CHUNK), jnp.int32),
            pltpu.VMEM((B_PER_W, DIM), jnp.float32),
            pltpu.VMEM((B_PER_W, DIM), jnp.float32),
            pltpu.VMEM((B_PER_W,), jnp.float32),
            pltpu.SemaphoreType.DMA,
        ],
    )
    return gather_dot(item_idx, item_table, u_rows)
